# Initial kernel scaffold; baseline (speedup 1.0000x reference)
import jax
import jax.numpy as jnp
from jax.experimental import pallas as pl


def kernel(positions, rcov, r4r2, c6_table, cn_ref, edge_index, types):
    raise NotImplementedError("write your pallas kernel here")


# speedup vs baseline: 57.7404x; 57.7404x over previous
import functools

import jax
import jax.numpy as jnp
from jax import lax
from jax.experimental import pallas as pl
from jax.experimental.pallas import tpu as pltpu
from jax.experimental.pallas import tpu_sc as plsc

N = 50000
E = 800000
Z = 95
M = 5
A1 = 0.4289
A2 = 4.4407
S6 = 1.0
S8 = 0.7875
CN_CUTOFF = 25.0
DISP_CUTOFF = 50.0

NC = 2
NS = 16
NW = NC * NS
C = 128
NPAD = 50176
SL = NPAD // NW
ECH = E // C
ACH = NPAD // C
CP = 32

_mesh = plsc.VectorSubcoreMesh(core_axis_name="c", subcore_axis_name="s")


def _i16():
    return lax.iota(jnp.int32, 16)


def _full16(v):
    return jnp.full((16,), v, jnp.int32)


def _rsqrt(x):
    y = plsc.bitcast(jnp.int32(0x5F3759DF) - (plsc.bitcast(x, jnp.int32) >> 1),
                     jnp.float32)
    for _ in range(3):
        y = y * (1.5 - 0.5 * x * y * y)
    return y


def _sqrt(x):
    return x * _rsqrt(x)


def _stage(hbm, spm, buf, s):
    for k in range(2):
        off = s * (2 * SL) + k * SL
        pltpu.sync_copy(hbm.at[pl.ds(off, SL)], buf)
        pltpu.sync_copy(buf, spm.at[pl.ds(off, SL)])


def _unstage(spm, hbm, buf, s):
    for k in range(2):
        off = s * (2 * SL) + k * SL
        pltpu.sync_copy(spm.at[pl.ds(off, SL)], buf)
        pltpu.sync_copy(buf, hbm.at[pl.ds(off, SL)])


def _zero_spmem(spm, zbuf, s):
    def _z(i, _):
        zbuf[pl.ds(i * 16, 16)] = jnp.zeros((16,), jnp.float32)
        return 0
    lax.fori_loop(0, SL // 16, _z, 0)
    for k in range(2):
        pltpu.sync_copy(zbuf, spm.at[pl.ds(s * (2 * SL) + k * SL, SL)])


def _cn_body(src_hbm, dst_hbm, px_hbm, py_hbm, pz_hbm, types_hbm, rcov_hbm,
             cn_a, cn_b, r_out,
             px_s, py_s, pz_s, types_s, cn_s,
             rcov_t, src_v, dst_v, tsrc_v, tdst_v,
             xA, yA, zA, xB, yB, zB, contrib_v, r_v, fbuf, ibuf):
    c = lax.axis_index("c")
    s = lax.axis_index("s")
    w = s * NC + c

    _stage(px_hbm, px_s, fbuf, s)
    _stage(py_hbm, py_s, fbuf, s)
    _stage(pz_hbm, pz_s, fbuf, s)
    _stage(types_hbm, types_s, ibuf, s)
    pltpu.sync_copy(rcov_hbm, rcov_t)
    _zero_spmem(cn_s, fbuf, s)
    plsc.subcore_barrier()

    nch = 195 + jnp.where(w < (ECH - 195 * NW), 1, 0)

    def _chunk(j, _):
        base = (w + NW * j) * C
        pltpu.sync_copy(src_hbm.at[pl.ds(base, C)], src_v)
        pltpu.sync_copy(dst_hbm.at[pl.ds(base, C)], dst_v)
        pltpu.sync_copy(px_s.at[src_v], xA)
        pltpu.sync_copy(py_s.at[src_v], yA)
        pltpu.sync_copy(pz_s.at[src_v], zA)
        pltpu.sync_copy(px_s.at[dst_v], xB)
        pltpu.sync_copy(py_s.at[dst_v], yB)
        pltpu.sync_copy(pz_s.at[dst_v], zB)
        pltpu.sync_copy(types_s.at[src_v], tsrc_v)
        pltpu.sync_copy(types_s.at[dst_v], tdst_v)
        for g in range(C // 16):
            sl = pl.ds(g * 16, 16)
            dx = xB[sl] - xA[sl]
            dy = yB[sl] - yA[sl]
            dz = zB[sl] - zA[sl]
            r2 = dx * dx + dy * dy + dz * dz
            r = _sqrt(r2 + 1e-12)
            zi = tsrc_v[sl]
            zj = tdst_v[sl]
            rc = plsc.load_gather(rcov_t, [zi]) + plsc.load_gather(rcov_t, [zj])
            cc = 1.0 / (1.0 + jnp.exp(-16.0 * (rc / r - 1.0)))
            cc = jnp.where(r < CN_CUTOFF, cc, 0.0)
            contrib_v[sl] = cc
            r_v[sl] = r
        pltpu.sync_copy(contrib_v, cn_s.at[src_v], add=True)
        pltpu.sync_copy(r_v, r_out.at[pl.ds(base, C)])
        return 0

    lax.fori_loop(0, nch, _chunk, 0)
    plsc.subcore_barrier()

    @pl.when(c == 0)
    def _():
        _unstage(cn_s, cn_a, fbuf, s)

    @pl.when(c == 1)
    def _():
        _unstage(cn_s, cn_b, fbuf, s)


_k1 = functools.partial(
    pl.kernel,
    out_type=(
        jax.ShapeDtypeStruct((NPAD,), jnp.float32),
        jax.ShapeDtypeStruct((NPAD,), jnp.float32),
        jax.ShapeDtypeStruct((E,), jnp.float32),
    ),
    mesh=_mesh,
    compiler_params=pltpu.CompilerParams(needs_layout_passes=False, use_tc_tiling_on_sc=False),
    scratch_types=[
        pltpu.VMEM_SHARED((NPAD,), jnp.float32),
        pltpu.VMEM_SHARED((NPAD,), jnp.float32),
        pltpu.VMEM_SHARED((NPAD,), jnp.float32),
        pltpu.VMEM_SHARED((NPAD,), jnp.int32),
        pltpu.VMEM_SHARED((NPAD,), jnp.float32),
        pltpu.VMEM((Z,), jnp.float32),
        pltpu.VMEM((C,), jnp.int32),
        pltpu.VMEM((C,), jnp.int32),
        pltpu.VMEM((C,), jnp.int32),
        pltpu.VMEM((C,), jnp.int32),
        pltpu.VMEM((C,), jnp.float32),
        pltpu.VMEM((C,), jnp.float32),
        pltpu.VMEM((C,), jnp.float32),
        pltpu.VMEM((C,), jnp.float32),
        pltpu.VMEM((C,), jnp.float32),
        pltpu.VMEM((C,), jnp.float32),
        pltpu.VMEM((C,), jnp.float32),
        pltpu.VMEM((C,), jnp.float32),
        pltpu.VMEM((SL,), jnp.float32),
        pltpu.VMEM((SL,), jnp.int32),
    ],
)(_cn_body)


def _atom_body(cn0_hbm, cn1_hbm, types_hbm, cnref_hbm, r4r2_hbm,
               v0, v1, v2, v3, v4, r4a,
               cnref_t, r4r2_t, cnA_v, cnB_v, t_v,
               o0, o1, o2, o3, o4, o5):
    c = lax.axis_index("c")
    s = lax.axis_index("s")
    w = s * NC + c
    pltpu.sync_copy(cnref_hbm, cnref_t)
    pltpu.sync_copy(r4r2_hbm, r4r2_t)

    nch = (ACH // NW) + jnp.where(w < (ACH - (ACH // NW) * NW), 1, 0)
    outs = (o0, o1, o2, o3, o4)
    vouts = (v0, v1, v2, v3, v4)

    def _chunk(j, _):
        base = (w + NW * j) * C
        pltpu.sync_copy(cn0_hbm.at[pl.ds(base, C)], cnA_v)
        pltpu.sync_copy(cn1_hbm.at[pl.ds(base, C)], cnB_v)
        pltpu.sync_copy(types_hbm.at[pl.ds(base, C)], t_v)
        for g in range(C // 16):
            sl = pl.ds(g * 16, 16)
            z = t_v[sl]
            cn = cnA_v[sl] + cnB_v[sl]
            zM = z * M
            es = []
            for a in range(M):
                ra = plsc.load_gather(cnref_t, [zM + a])
                d = cn - ra
                es.append(-4.0 * d * d)
            mx = es[0]
            for a in range(1, M):
                mx = jnp.maximum(mx, es[a])
            ws = [jnp.exp(e - mx) for e in es]
            ssum = ws[0]
            for a in range(1, M):
                ssum = ssum + ws[a]
            inv = 1.0 / ssum
            for a in range(M):
                outs[a][sl] = ws[a] * inv
            o5[sl] = plsc.load_gather(r4r2_t, [z])
        for a in range(M):
            pltpu.sync_copy(outs[a], vouts[a].at[pl.ds(base, C)])
        pltpu.sync_copy(o5, r4a.at[pl.ds(base, C)])
        return 0

    lax.fori_loop(0, nch, _chunk, 0)


_k2 = functools.partial(
    pl.kernel,
    out_type=tuple(jax.ShapeDtypeStruct((NPAD,), jnp.float32)
                   for _ in range(6)),
    mesh=_mesh,
    compiler_params=pltpu.CompilerParams(needs_layout_passes=False, use_tc_tiling_on_sc=False),
    scratch_types=[
        pltpu.VMEM((Z * M,), jnp.float32),
        pltpu.VMEM((Z,), jnp.float32),
        pltpu.VMEM((C,), jnp.float32),
        pltpu.VMEM((C,), jnp.float32),
        pltpu.VMEM((C,), jnp.int32),
        pltpu.VMEM((C,), jnp.float32),
        pltpu.VMEM((C,), jnp.float32),
        pltpu.VMEM((C,), jnp.float32),
        pltpu.VMEM((C,), jnp.float32),
        pltpu.VMEM((C,), jnp.float32),
        pltpu.VMEM((C,), jnp.float32),
    ],
)(_atom_body)


def _pair_body(src_hbm, dst_hbm, r_hbm, v_hbm0, v_hbm1, v_hbm2, v_hbm3,
               v_hbm4, r4_hbm, types_hbm, c6_hbm,
               pa_a, pa_b,
               v0_s, v1_s, v2_s, v3_s, v4_s, r4_s, types_s, pa_s,
               src_v, dst_v, r_v, tsrc_v, tdst_v,
               viA0, viA1, viA2, viA3, viA4, r4A,
               viB0, viB1, viB2, viB3, viB4, r4B,
               c6i_v, c6rows, eh_v, fbuf, ibuf):
    c = lax.axis_index("c")
    s = lax.axis_index("s")
    w = s * NC + c

    vS = (v0_s, v1_s, v2_s, v3_s, v4_s)
    vH = (v_hbm0, v_hbm1, v_hbm2, v_hbm3, v_hbm4)
    for a in range(M):
        _stage(vH[a], vS[a], fbuf, s)
    _stage(r4_hbm, r4_s, fbuf, s)
    _stage(types_hbm, types_s, ibuf, s)
    _zero_spmem(pa_s, fbuf, s)
    plsc.subcore_barrier()

    nch = 195 + jnp.where(w < (ECH - 195 * NW), 1, 0)
    vA = (viA0, viA1, viA2, viA3, viA4)
    vB = (viB0, viB1, viB2, viB3, viB4)

    def _chunk(j, _):
        base = (w + NW * j) * C
        pltpu.sync_copy(src_hbm.at[pl.ds(base, C)], src_v)
        pltpu.sync_copy(dst_hbm.at[pl.ds(base, C)], dst_v)
        pltpu.sync_copy(r_hbm.at[pl.ds(base, C)], r_v)
        pltpu.sync_copy(types_s.at[src_v], tsrc_v)
        pltpu.sync_copy(types_s.at[dst_v], tdst_v)
        for a in range(M):
            pltpu.sync_copy(vS[a].at[src_v], vA[a])
            pltpu.sync_copy(vS[a].at[dst_v], vB[a])
        pltpu.sync_copy(r4_s.at[src_v], r4A)
        pltpu.sync_copy(r4_s.at[dst_v], r4B)
        for g in range(C // 16):
            sl = pl.ds(g * 16, 16)
            c6i_v[sl] = tsrc_v[sl] * Z + tdst_v[sl]
        pltpu.sync_copy(c6_hbm.at[c6i_v], c6rows)
        for g in range(C // 16):
            sl = pl.ds(g * 16, 16)
            rows = _i16() + g * 16
            c6acc = jnp.zeros((16,), jnp.float32)
            for a in range(M):
                inner = jnp.zeros((16,), jnp.float32)
                for b in range(M):
                    cab = plsc.load_gather(c6rows, [rows, _full16(a * M + b)])
                    inner = inner + vB[b][sl] * cab
                c6acc = c6acc + vA[a][sl] * inner
            qq = 3.0 * r4A[sl] * r4B[sl]
            r = r_v[sl]
            r2 = r * r
            r6 = r2 * r2 * r2
            r8 = r6 * r2
            r0 = A1 * _sqrt(qq) + A2
            r0_2 = r0 * r0
            r0_6 = r0_2 * r0_2 * r0_2
            r0_8 = r0_6 * r0_2
            e6 = c6acc / (r6 + r0_6)
            e8 = (c6acc * qq) / (r8 + r0_8)
            ep = -0.5 * (S6 * e6 + S8 * e8)
            ep = jnp.where(r < DISP_CUTOFF, ep, 0.0)
            eh_v[sl] = 0.5 * ep
        pltpu.sync_copy(eh_v, pa_s.at[src_v], add=True)
        pltpu.sync_copy(eh_v, pa_s.at[dst_v], add=True)
        return 0

    lax.fori_loop(0, nch, _chunk, 0)
    plsc.subcore_barrier()

    @pl.when(c == 0)
    def _():
        _unstage(pa_s, pa_a, fbuf, s)

    @pl.when(c == 1)
    def _():
        _unstage(pa_s, pa_b, fbuf, s)


_k3 = functools.partial(
    pl.kernel,
    out_type=(
        jax.ShapeDtypeStruct((NPAD,), jnp.float32),
        jax.ShapeDtypeStruct((NPAD,), jnp.float32),
    ),
    mesh=_mesh,
    compiler_params=pltpu.CompilerParams(needs_layout_passes=False, use_tc_tiling_on_sc=False),
    scratch_types=(
        [pltpu.VMEM_SHARED((NPAD,), jnp.float32) for _ in range(6)]
        + [pltpu.VMEM_SHARED((NPAD,), jnp.int32),
           pltpu.VMEM_SHARED((NPAD,), jnp.float32),
           pltpu.VMEM((C,), jnp.int32),
           pltpu.VMEM((C,), jnp.int32),
           pltpu.VMEM((C,), jnp.float32),
           pltpu.VMEM((C,), jnp.int32),
           pltpu.VMEM((C,), jnp.int32)]
        + [pltpu.VMEM((C,), jnp.float32) for _ in range(12)]
        + [pltpu.VMEM((C,), jnp.int32),
           pltpu.VMEM((C, CP), jnp.float32),
           pltpu.VMEM((C,), jnp.float32),
           pltpu.VMEM((SL,), jnp.float32),
           pltpu.VMEM((SL,), jnp.int32)]
    ),
)(_pair_body)


def kernel(positions, rcov, r4r2, c6_table, cn_ref, edge_index, types):
    src = edge_index[0]
    dst = edge_index[1]
    pxp = jnp.zeros((NPAD,), jnp.float32).at[:N].set(positions[:, 0])
    pyp = jnp.zeros((NPAD,), jnp.float32).at[:N].set(positions[:, 1])
    pzp = jnp.zeros((NPAD,), jnp.float32).at[:N].set(positions[:, 2])
    types_pad = jnp.zeros((NPAD,), jnp.int32).at[:N].set(types)
    c6p = jnp.pad(c6_table.reshape(Z * Z, M * M), ((0, 0), (0, CP - M * M)))
    cnref_f = cn_ref.reshape(Z * M)

    cn_a, cn_b, r_e = _k1(src, dst, pxp, pyp, pzp, types_pad, rcov)
    v0, v1, v2, v3, v4, r4a = _k2(cn_a, cn_b, types_pad, cnref_f, r4r2)
    pa_a, pa_b = _k3(src, dst, r_e, v0, v1, v2, v3, v4, r4a, types_pad, c6p)
    return (pa_a + pa_b)[:N]
